# Initial kernel scaffold; baseline (speedup 1.0000x reference)
import jax
import jax.numpy as jnp
from jax.experimental import pallas as pl


def kernel(hidden_states, W):
    raise NotImplementedError("write your pallas kernel here")


# speedup vs baseline: 1.2747x; 1.2747x over previous
import jax
import jax.numpy as jnp
from jax import lax
from jax.experimental import pallas as pl

_NUM_EXPERTS = 64
_TOP_K = 8
_TOKEN_BLOCK = 512


def _moe_gate_body(x_ref, wt_ref, probs_ref, idx_ref, wts_ref, counts_ref):
    x = x_ref[...]
    wt = wt_ref[...]
    logits = jnp.dot(x, wt, preferred_element_type=jnp.float32)

    m = jnp.max(logits, axis=-1, keepdims=True)
    e = jnp.exp(logits - m)
    denom = jnp.sum(e, axis=-1, keepdims=True)
    probs = e / denom
    probs_ref[...] = probs

    tb, n_exp = probs.shape
    lane = lax.broadcasted_iota(jnp.int32, (tb, n_exp), 1)
    work = probs
    onehot_sum = jnp.zeros((tb, n_exp), jnp.float32)
    idx_cols = []
    val_cols = []
    for _ in range(_TOP_K):
        mx = jnp.max(work, axis=-1, keepdims=True)
        sel = jnp.min(jnp.where(work == mx, lane, n_exp), axis=-1,
                      keepdims=True)
        onehot = lane == sel
        idx_cols.append(sel)
        val_cols.append(mx)
        onehot_sum = onehot_sum + onehot.astype(jnp.float32)
        work = jnp.where(onehot, -1.0, work)

    idx = jnp.concatenate(idx_cols, axis=-1)
    vals = jnp.concatenate(val_cols, axis=-1)
    idx_ref[...] = idx
    wts_ref[...] = vals / jnp.sum(vals, axis=-1, keepdims=True)

    counts = jnp.sum(onehot_sum, axis=0, keepdims=True)

    @pl.when(pl.program_id(0) == 0)
    def _init():
        counts_ref[...] = jnp.zeros_like(counts_ref)

    counts_ref[...] += counts


def kernel(hidden_states, W):
    b, s, h = hidden_states.shape
    n_exp, _ = W.shape
    tokens = b * s
    tb = _TOKEN_BLOCK
    x = hidden_states.reshape(tokens, h)

    probs, idx, wts, counts = pl.pallas_call(
        _moe_gate_body,
        grid=(tokens // tb,),
        in_specs=[
            pl.BlockSpec((tb, h), lambda i: (i, 0)),
            pl.BlockSpec((h, n_exp), lambda i: (0, 0)),
        ],
        out_specs=[
            pl.BlockSpec((tb, n_exp), lambda i: (i, 0)),
            pl.BlockSpec((tb, _TOP_K), lambda i: (i, 0)),
            pl.BlockSpec((tb, _TOP_K), lambda i: (i, 0)),
            pl.BlockSpec((1, n_exp), lambda i: (0, 0)),
        ],
        out_shape=[
            jax.ShapeDtypeStruct((tokens, n_exp), jnp.float32),
            jax.ShapeDtypeStruct((tokens, _TOP_K), jnp.int32),
            jax.ShapeDtypeStruct((tokens, _TOP_K), jnp.float32),
            jax.ShapeDtypeStruct((1, n_exp), jnp.float32),
        ],
    )(x, W.T)

    expert_indices = idx.reshape(b, s, _TOP_K)
    routing_weights = wts.reshape(b, s, _TOP_K)
    expert_counts = counts.reshape(n_exp).astype(jnp.int64)
    router_probs = probs.reshape(b, s, n_exp)
    return (expert_indices, routing_weights, expert_counts, router_probs)


# speedup vs baseline: 1.5085x; 1.1834x over previous
import jax
import jax.numpy as jnp
from jax import lax
from jax.experimental import pallas as pl

_NUM_EXPERTS = 64
_TOP_K = 8
_TOKEN_BLOCK = 512


def _moe_gate_body(x_ref, wt_ref, probs_ref, idx_ref, wts_ref, counts_ref):
    x = x_ref[...]
    wt = wt_ref[...]
    logits = jnp.dot(x, wt, preferred_element_type=jnp.float32)

    m = jnp.max(logits, axis=-1, keepdims=True)
    e = jnp.exp(logits - m)
    denom = jnp.sum(e, axis=-1, keepdims=True)
    probs = e / denom
    probs_ref[...] = probs

    tb, n_exp = probs.shape
    lane = lax.broadcasted_iota(jnp.int32, (tb, n_exp), 1)
    bits = lax.bitcast_convert_type(probs, jnp.int32)
    keys = lax.bitcast_convert_type(
        (bits & jnp.int32(~63)) | (jnp.int32(n_exp - 1) - lane), jnp.float32)
    work = keys
    key_cols = []
    for _ in range(_TOP_K):
        mx = jnp.max(work, axis=-1, keepdims=True)
        key_cols.append(mx)
        work = jnp.where(work == mx, -1.0, work)

    mxs = jnp.concatenate(key_cols, axis=-1)
    mbits = lax.bitcast_convert_type(mxs, jnp.int32)
    idx_ref[...] = jnp.int32(n_exp - 1) - (mbits & jnp.int32(63))
    vals = lax.bitcast_convert_type(mbits & jnp.int32(~63), jnp.float32)
    wts_ref[...] = vals / jnp.sum(vals, axis=-1, keepdims=True)

    selected = jnp.where(work < 0.0, 1.0, 0.0)
    counts = jnp.sum(selected, axis=0, keepdims=True)

    @pl.when(pl.program_id(0) == 0)
    def _init():
        counts_ref[...] = jnp.zeros_like(counts_ref)

    counts_ref[...] += counts


def kernel(hidden_states, W):
    b, s, h = hidden_states.shape
    n_exp, _ = W.shape
    tokens = b * s
    tb = _TOKEN_BLOCK
    x = hidden_states.reshape(tokens, h)

    probs, idx, wts, counts = pl.pallas_call(
        _moe_gate_body,
        grid=(tokens // tb,),
        in_specs=[
            pl.BlockSpec((tb, h), lambda i: (i, 0)),
            pl.BlockSpec((h, n_exp), lambda i: (0, 0)),
        ],
        out_specs=[
            pl.BlockSpec((tb, n_exp), lambda i: (i, 0)),
            pl.BlockSpec((tb, _TOP_K), lambda i: (i, 0)),
            pl.BlockSpec((tb, _TOP_K), lambda i: (i, 0)),
            pl.BlockSpec((1, n_exp), lambda i: (0, 0)),
        ],
        out_shape=[
            jax.ShapeDtypeStruct((tokens, n_exp), jnp.float32),
            jax.ShapeDtypeStruct((tokens, _TOP_K), jnp.int32),
            jax.ShapeDtypeStruct((tokens, _TOP_K), jnp.float32),
            jax.ShapeDtypeStruct((1, n_exp), jnp.float32),
        ],
    )(x, W.T)

    expert_indices = idx.reshape(b, s, _TOP_K)
    routing_weights = wts.reshape(b, s, _TOP_K)
    expert_counts = counts.reshape(n_exp).astype(jnp.int64)
    router_probs = probs.reshape(b, s, n_exp)
    return (expert_indices, routing_weights, expert_counts, router_probs)


# speedup vs baseline: 1.6334x; 1.0828x over previous
import jax
import jax.numpy as jnp
from jax import lax
from jax.experimental import pallas as pl

_NUM_EXPERTS = 64
_TOP_K = 8
_TOKEN_BLOCK = 1024


def _moe_gate_body(x_ref, wt_ref, probs_ref, idx_ref, wts_ref, counts_ref):
    x = x_ref[...]
    wt = wt_ref[...]
    logits = jnp.dot(x, wt, preferred_element_type=jnp.float32)

    m = jnp.max(logits, axis=-1, keepdims=True)
    e = jnp.exp(logits - m)
    denom = jnp.sum(e, axis=-1, keepdims=True)
    probs = e / denom
    probs_ref[...] = probs

    tb, n_exp = probs.shape
    lane = lax.broadcasted_iota(jnp.int32, (tb, n_exp), 1)
    bits = lax.bitcast_convert_type(probs, jnp.int32)
    keys = lax.bitcast_convert_type(
        (bits & jnp.int32(~63)) | (jnp.int32(n_exp - 1) - lane), jnp.float32)
    work = keys
    key_cols = []
    for _ in range(_TOP_K):
        mx = jnp.max(work, axis=-1, keepdims=True)
        key_cols.append(mx)
        work = jnp.where(work == mx, -1.0, work)

    mxs = jnp.concatenate(key_cols, axis=-1)
    mbits = lax.bitcast_convert_type(mxs, jnp.int32)
    idx_ref[...] = jnp.int32(n_exp - 1) - (mbits & jnp.int32(63))
    vals = lax.bitcast_convert_type(mbits & jnp.int32(~63), jnp.float32)
    wts_ref[...] = vals / jnp.sum(vals, axis=-1, keepdims=True)

    selected = jnp.where(work < 0.0, 1.0, 0.0)
    counts = jnp.sum(selected, axis=0, keepdims=True)

    @pl.when(pl.program_id(0) == 0)
    def _init():
        counts_ref[...] = jnp.zeros_like(counts_ref)

    counts_ref[...] += counts


def kernel(hidden_states, W):
    b, s, h = hidden_states.shape
    n_exp, _ = W.shape
    tokens = b * s
    tb = _TOKEN_BLOCK
    x = hidden_states.reshape(tokens, h)

    probs, idx, wts, counts = pl.pallas_call(
        _moe_gate_body,
        grid=(tokens // tb,),
        in_specs=[
            pl.BlockSpec((tb, h), lambda i: (i, 0)),
            pl.BlockSpec((h, n_exp), lambda i: (0, 0)),
        ],
        out_specs=[
            pl.BlockSpec((tb, n_exp), lambda i: (i, 0)),
            pl.BlockSpec((tb, _TOP_K), lambda i: (i, 0)),
            pl.BlockSpec((tb, _TOP_K), lambda i: (i, 0)),
            pl.BlockSpec((1, n_exp), lambda i: (0, 0)),
        ],
        out_shape=[
            jax.ShapeDtypeStruct((tokens, n_exp), jnp.float32),
            jax.ShapeDtypeStruct((tokens, _TOP_K), jnp.int32),
            jax.ShapeDtypeStruct((tokens, _TOP_K), jnp.float32),
            jax.ShapeDtypeStruct((1, n_exp), jnp.float32),
        ],
    )(x, W.T)

    expert_indices = idx.reshape(b, s, _TOP_K)
    routing_weights = wts.reshape(b, s, _TOP_K)
    expert_counts = counts.reshape(n_exp).astype(jnp.int64)
    router_probs = probs.reshape(b, s, n_exp)
    return (expert_indices, routing_weights, expert_counts, router_probs)
